# baseline (speedup 1.0000x reference)
import jax
import jax.numpy as jnp
from jax.experimental import pallas as pl
from jax.experimental.pallas import tpu as pltpu

_BT = 512
_NSTREAMS = 4


def _fbe_block(*refs):
    flag_refs = refs[:_NSTREAMS]
    emba_ref = refs[_NSTREAMS]
    out_ref = refs[_NSTREAMS + 1]
    emba = emba_ref[:]
    d = out_ref.shape[1]
    for j, f in enumerate(flag_refs):
        mask = (f[:] > 0.5).astype(jnp.float32)
        acc = jnp.dot(mask, emba,
                      preferred_element_type=jnp.float32)
        sums = acc[:, :d]
        counts = acc[:, d:d + 1]
        out_ref[j * _BT:(j + 1) * _BT, :] = sums / jnp.maximum(counts, 1.0)


def kernel(flags_matrix, emb):
    t, k = flags_matrix.shape
    k2, d = emb.shape
    emb_aug = jnp.concatenate([emb, jnp.ones((k2, 1), jnp.float32)], axis=1)
    rows_per_step = _BT * _NSTREAMS
    grid = t // rows_per_step
    in_specs = [
        pl.BlockSpec((_BT, k), lambda i, j=j: (i * _NSTREAMS + j, 0))
        for j in range(_NSTREAMS)
    ] + [pl.BlockSpec((k2, d + 1), lambda i: (0, 0))]
    return pl.pallas_call(
        _fbe_block,
        grid=(grid,),
        in_specs=in_specs,
        out_specs=pl.BlockSpec((rows_per_step, d), lambda i: (i, 0)),
        out_shape=jax.ShapeDtypeStruct((t, d), jnp.float32),
        compiler_params=pltpu.CompilerParams(
            dimension_semantics=("arbitrary",),
        ),
    )(*([flags_matrix] * _NSTREAMS), emb_aug)
